# baseline (speedup 1.0000x reference)
import functools

import jax
import jax.numpy as jnp
from jax import lax
from jax.experimental import pallas as pl
from jax.experimental.pallas import tpu as pltpu
from jax.experimental.pallas import tpu_sc as plsc

N = 10000
E = 320000
D = 128

NC = 2
NS = 16
NW = NC * NS
K = 64
C = 160
GREAL = E // K
SROW = 2 * K
NB = 4
NPAD = 10112
RPT = NPAD // NS
BLK = 1000


def _mlp1_body(x_ref, w_ref, b_ref, o_ref):
    acc = jnp.dot(x_ref[...], w_ref[...], preferred_element_type=jnp.float32)
    o_ref[...] = jnp.maximum(acc + b_ref[...], 0.0)


def _mlp1(x, W, b):
    return pl.pallas_call(
        _mlp1_body,
        grid=(N // BLK,),
        in_specs=[
            pl.BlockSpec((BLK, D), lambda i: (i, 0)),
            pl.BlockSpec((D, D), lambda i: (0, 0)),
            pl.BlockSpec((1, D), lambda i: (0, 0)),
        ],
        out_specs=pl.BlockSpec((BLK, D), lambda i: (i, 0)),
        out_shape=jax.ShapeDtypeStruct((N, D), jnp.float32),
    )(x, W, b)


def _mlp2_body(x_ref, p0_ref, p1_ref, w_ref, b_ref, o_ref):
    s = x_ref[...] + p0_ref[...] + p1_ref[...]
    acc = jnp.dot(s, w_ref[...], preferred_element_type=jnp.float32)
    o_ref[...] = jnp.maximum(acc + b_ref[...], 0.0)


def _mlp2(x, p0, p1, W, b):
    return pl.pallas_call(
        _mlp2_body,
        grid=(N // BLK,),
        in_specs=[
            pl.BlockSpec((BLK, D), lambda i: (i, 0)),
            pl.BlockSpec((BLK, D), lambda i: (i, 0)),
            pl.BlockSpec((BLK, D), lambda i: (i, 0)),
            pl.BlockSpec((D, D), lambda i: (0, 0)),
            pl.BlockSpec((1, D), lambda i: (0, 0)),
        ],
        out_specs=pl.BlockSpec((BLK, D), lambda i: (i, 0)),
        out_shape=jax.ShapeDtypeStruct((N, D), jnp.float32),
    )(x, p0, p1, W, b)


def _lane_bcast(vec, lane):
    idx = jnp.full((16, 1), lane, jnp.int32)
    dnums = lax.GatherDimensionNumbers(
        offset_dims=(), collapsed_slice_dims=(0,), start_index_map=(0,))
    return lax.gather(vec, idx, dnums, (1,),
                      mode=lax.GatherScatterMode.PROMISE_IN_BOUNDS)


_MESH = plsc.VectorSubcoreMesh(core_axis_name="c", subcore_axis_name="s")


@functools.partial(
    pl.kernel,
    mesh=_MESH,
    compiler_params=pltpu.CompilerParams(use_tc_tiling_on_sc=False),
    out_type=[jax.ShapeDtypeStruct((NPAD, D), jnp.float32),
              jax.ShapeDtypeStruct((NPAD, D), jnp.float32)],
    scratch_types=[
        pltpu.VMEM_SHARED((NPAD, D), jnp.float32),
        pltpu.VMEM((C // 2, SROW), jnp.int32),
        pltpu.VMEM((1, K), jnp.int32),
        pltpu.VMEM((1, K), jnp.int32),
        pltpu.VMEM((1, K), jnp.int32),
        pltpu.VMEM((1, K), jnp.int32),
        pltpu.VMEM((1, K), jnp.float32),
        pltpu.VMEM((1, K), jnp.float32),
        pltpu.VMEM((K, D // 2), jnp.int32),
        pltpu.VMEM((K, D // 2), jnp.int32),
        pltpu.VMEM((K, D // 2), jnp.int32),
        pltpu.VMEM((K, D // 2), jnp.int32),
        pltpu.VMEM((K, D), jnp.float32),
        pltpu.VMEM((K, D), jnp.float32),
        pltpu.SemaphoreType.DMA,
        pltpu.SemaphoreType.DMA,
        pltpu.SemaphoreType.DMA,
        pltpu.SemaphoreType.DMA,
        pltpu.SemaphoreType.DMA,
        pltpu.SemaphoreType.DMA,
        pltpu.SemaphoreType.DMA,
        pltpu.SemaphoreType.DMA,
        pltpu.SemaphoreType.DMA,
        pltpu.SemaphoreType.DMA,
        pltpu.SemaphoreType.DMA,
        pltpu.SemaphoreType.DMA,
    ],
)
def _edge_agg(h_hbm, src_hbm, dst_hbm, w_hbm, out0_hbm, out1_hbm,
              agg_sh, src_v, dst0, dst1, dst2, dst3, w0, w1,
              rbf0, rbf1, rbf2, rbf3, sb0, sb1,
              gs0, gs1, gs2, gs3, ds0, ds1, ds2, ds3, ws0, ws1,
              ss0, ss1):
    cid = lax.axis_index("c")
    tid = lax.axis_index("s")
    wid = cid * NS + tid
    g0 = wid * C

    dstb = (dst0, dst1, dst2, dst3)
    wb_ = (w0, w1)
    rbf = (rbf0, rbf1, rbf2, rbf3)
    sbuf = (sb0, sb1)
    gsem = (gs0, gs1, gs2, gs3)
    dsem = (ds0, ds1, ds2, ds3)
    wsem = (ws0, ws1)
    ssem = (ss0, ss1)

    @pl.when(wid < NW - 1)
    def _stage_full():
        pltpu.sync_copy(src_hbm.at[pl.ds(wid * (C // 2), C // 2)], src_v)

    @pl.when(wid == NW - 1)
    def _stage_tail():
        nrow = E // SROW - (NW - 1) * (C // 2)
        pltpu.sync_copy(src_hbm.at[pl.ds((NW - 1) * (C // 2), nrow)],
                        src_v.at[pl.ds(0, nrow)])

    class _Dma:
        def __init__(self, src, dst, sem, add=False):
            self.args = (src, dst, sem)
            self.add = add

        def start(self):
            pltpu.async_copy(*self.args, add=self.add)

        def wait(self):
            pltpu.make_async_copy(*self.args).wait()

    def _gather(b, row, half):
        idx = src_v.at[row, pl.ds(half * K, K)]
        return _Dma(h_hbm.at[idx], rbf[b], gsem[b])

    def _dst_cp(b, c):
        return _Dma(dst_hbm.at[g0 + c], dstb[b], dsem[b])

    def _w_cp(b, c):
        return _Dma(w_hbm.at[g0 + c], wb_[b], wsem[b])

    def _scat(p, bd):
        return _Dma(sbuf[p], agg_sh.at[dstb[bd].at[0]], ssem[p], add=True)

    zero = jnp.zeros((16,), jnp.float32)

    def _zrow(r, carry):
        for j in range(D // 16):
            sb0[r, pl.ds(j * 16, 16)] = zero
        return carry

    lax.fori_loop(0, K, _zrow, 0)
    base = tid * RPT
    for k in range(9):
        pltpu.sync_copy(sb0, agg_sh.at[pl.ds(base + k * K, K)])
    pltpu.sync_copy(sb0.at[pl.ds(0, RPT - 9 * K)],
                    agg_sh.at[pl.ds(base + 9 * K, RPT - 9 * K)])
    plsc.subcore_barrier()

    _dst_cp(0, 0).start()
    _dst_cp(1, 1).start()
    _w_cp(0, 0).start()
    _w_cp(1, 1).start()
    _gather(0, 0, 0).start()
    _gather(1, 0, 1).start()

    def _chunk4(c4, carry):
        for sub in range(NB):
            b = sub
            p = sub % 2
            c = c4 * NB + sub
            b2 = (sub + 2) % NB
            row2 = (c + 2) // 2
            half2 = (sub + 2) % 2

            @pl.when((c >= 2) & (g0 + c < GREAL))
            def _drain(p=p, b2=b2):
                _scat(p, b2).wait()

            @pl.when((c < C - 2) & (g0 + c + 2 < GREAL))
            def _issue(b2=b2, c=c, row2=row2, half2=half2):
                _dst_cp(b2, c + 2).start()
                _gather(b2, row2, half2).start()

            @pl.when(g0 + c < GREAL)
            def _work(b=b, c=c, sub=sub, p=p):
                _gather(b, 0, 0).wait()
                _w_cp(p, c).wait()

                def _grp(g, carry2, b=b, p=p):
                    wv = wb_[p][0, pl.ds(pl.multiple_of(g * 16, 16), 16)]
                    base2 = g * 16
                    for e in range(16):
                        wvb = _lane_bcast(wv, e)
                        r = base2 + e
                        for j in range(D // 32):
                            u32 = rbf[b][r, pl.ds(j * 16, 16)]
                            lo = lax.bitcast_convert_type(
                                u32 << 16, jnp.float32)
                            hi = lax.bitcast_convert_type(
                                u32 & jnp.int32(-65536), jnp.float32)
                            sbuf[p][r, pl.ds(j * 32, 16)] = lo * wvb
                            sbuf[p][r, pl.ds(j * 32 + 16, 16)] = hi * wvb
                    return carry2

                lax.fori_loop(0, K // 16, _grp, 0)

                @pl.when((c < C - 2) & (g0 + c + 2 < GREAL))
                def _issue_w(c=c, p=p):
                    _w_cp(p, c + 2).start()

                _dst_cp(b, c).wait()
                _scat(p, b).start()
        return carry

    lax.fori_loop(0, C // NB, _chunk4, 0)
    _scat(0, 2).wait()
    _scat(1, 3).wait()
    plsc.subcore_barrier()

    @pl.when(cid == 0)
    def _wb0():
        for k in range(9):
            off = base + k * K
            pltpu.sync_copy(agg_sh.at[pl.ds(off, K)], sb0)
            pltpu.sync_copy(sb0, out0_hbm.at[pl.ds(off, K)])
        tail = RPT - 9 * K
        off = base + 9 * K
        pltpu.sync_copy(agg_sh.at[pl.ds(off, tail)], sb0.at[pl.ds(0, tail)])
        pltpu.sync_copy(sb0.at[pl.ds(0, tail)], out0_hbm.at[pl.ds(off, tail)])

    @pl.when(cid == 1)
    def _wb1():
        for k in range(9):
            off = base + k * K
            pltpu.sync_copy(agg_sh.at[pl.ds(off, K)], sb0)
            pltpu.sync_copy(sb0, out1_hbm.at[pl.ds(off, K)])
        tail = RPT - 9 * K
        off = base + 9 * K
        pltpu.sync_copy(agg_sh.at[pl.ds(off, tail)], sb0.at[pl.ds(0, tail)])
        pltpu.sync_copy(sb0.at[pl.ds(0, tail)], out1_hbm.at[pl.ds(off, tail)])


def kernel(x, edge_index, edge_weight, W1, b1, W2, b2):
    src = edge_index[0].astype(jnp.int32).reshape(E // SROW, SROW)
    dst = edge_index[1].astype(jnp.int32).reshape(GREAL, 1, K)
    w = edge_weight.astype(jnp.float32).reshape(GREAL, 1, K)

    h = _mlp1(x, W1, b1.reshape(1, D))
    perm = (jnp.arange(D) // 32 * 32 + (jnp.arange(D) % 32 % 2) * 16
            + jnp.arange(D) % 32 // 2)
    hb = jnp.take(h, perm, axis=1).astype(jnp.bfloat16)
    hb32 = lax.bitcast_convert_type(hb.reshape(N, D // 2, 2), jnp.int32)
    p0, p1 = _edge_agg(hb32, src, dst, w)
    out = _mlp2(x, p0, p1, W2, b2.reshape(1, D))
    return out


# speedup vs baseline: 2.3726x; 2.3726x over previous
import functools

import jax
import jax.numpy as jnp
from jax import lax
from jax.experimental import pallas as pl
from jax.experimental.pallas import tpu as pltpu
from jax.experimental.pallas import tpu_sc as plsc

N = 10000
E = 320000
D = 128

NC = 2
NS = 16
NW = NC * NS
K = 64
C = 160
GREAL = E // K
SROW = 2 * K
NB = 4
NPAD = 10112
RPT = NPAD // NS
BLK = 1000


def _mlp1_body(x_ref, w_ref, b_ref, o_ref):
    acc = jnp.dot(x_ref[...], w_ref[...], preferred_element_type=jnp.float32)
    o_ref[...] = jnp.maximum(acc + b_ref[...], 0.0)


def _mlp1(x, W, b):
    return pl.pallas_call(
        _mlp1_body,
        grid=(N // BLK,),
        in_specs=[
            pl.BlockSpec((BLK, D), lambda i: (i, 0)),
            pl.BlockSpec((D, D), lambda i: (0, 0)),
            pl.BlockSpec((1, D), lambda i: (0, 0)),
        ],
        out_specs=pl.BlockSpec((BLK, D), lambda i: (i, 0)),
        out_shape=jax.ShapeDtypeStruct((N, D), jnp.float32),
    )(x, W, b)


def _mlp2_body(x_ref, p0_ref, p1_ref, w_ref, b_ref, o_ref):
    s = x_ref[...] + p0_ref[...] + p1_ref[...]
    acc = jnp.dot(s, w_ref[...], preferred_element_type=jnp.float32)
    o_ref[...] = jnp.maximum(acc + b_ref[...], 0.0)


def _mlp2(x, p0, p1, W, b):
    return pl.pallas_call(
        _mlp2_body,
        grid=(N // BLK,),
        in_specs=[
            pl.BlockSpec((BLK, D), lambda i: (i, 0)),
            pl.BlockSpec((BLK, D), lambda i: (i, 0)),
            pl.BlockSpec((BLK, D), lambda i: (i, 0)),
            pl.BlockSpec((D, D), lambda i: (0, 0)),
            pl.BlockSpec((1, D), lambda i: (0, 0)),
        ],
        out_specs=pl.BlockSpec((BLK, D), lambda i: (i, 0)),
        out_shape=jax.ShapeDtypeStruct((N, D), jnp.float32),
    )(x, p0, p1, W, b)


def _lane_bcast(vec, lane):
    idx = jnp.full((16, 1), lane, jnp.int32)
    dnums = lax.GatherDimensionNumbers(
        offset_dims=(), collapsed_slice_dims=(0,), start_index_map=(0,))
    return lax.gather(vec, idx, dnums, (1,),
                      mode=lax.GatherScatterMode.PROMISE_IN_BOUNDS)


_MESH = plsc.VectorSubcoreMesh(core_axis_name="c", subcore_axis_name="s")


@functools.partial(
    pl.kernel,
    mesh=_MESH,
    out_type=[jax.ShapeDtypeStruct((NPAD, D), jnp.float32),
              jax.ShapeDtypeStruct((NPAD, D), jnp.float32)],
    scratch_types=[
        pltpu.VMEM_SHARED((NPAD, D), jnp.float32),
        pltpu.VMEM((C * K,), jnp.int32),
        pltpu.VMEM((K,), jnp.int32),
        pltpu.VMEM((K,), jnp.int32),
        pltpu.VMEM((K,), jnp.int32),
        pltpu.VMEM((K,), jnp.int32),
        pltpu.VMEM((K,), jnp.float32),
        pltpu.VMEM((K,), jnp.float32),
        pltpu.VMEM((K, D), jnp.float32),
        pltpu.VMEM((K, D), jnp.float32),
        pltpu.VMEM((K, D), jnp.float32),
        pltpu.VMEM((K, D), jnp.float32),
        pltpu.SemaphoreType.DMA,
        pltpu.SemaphoreType.DMA,
        pltpu.SemaphoreType.DMA,
        pltpu.SemaphoreType.DMA,
        pltpu.SemaphoreType.DMA,
        pltpu.SemaphoreType.DMA,
        pltpu.SemaphoreType.DMA,
        pltpu.SemaphoreType.DMA,
        pltpu.SemaphoreType.DMA,
        pltpu.SemaphoreType.DMA,
        pltpu.SemaphoreType.DMA,
        pltpu.SemaphoreType.DMA,
        pltpu.SemaphoreType.DMA,
        pltpu.SemaphoreType.DMA,
    ],
)
def _edge_agg(h_hbm, src_hbm, dst_hbm, w_hbm, out0_hbm, out1_hbm,
              agg_sh, src_v, dst0, dst1, dst2, dst3, w0, w1,
              rows0, rows1, rows2, rows3,
              gs0, gs1, gs2, gs3, ds0, ds1, ds2, ds3, ws0, ws1,
              ss0, ss1, ss2, ss3):
    cid = lax.axis_index("c")
    tid = lax.axis_index("s")
    wid = cid * NS + tid
    g0 = wid * C

    dstb = (dst0, dst1, dst2, dst3)
    wb_ = (w0, w1)
    rows = (rows0, rows1, rows2, rows3)
    gsem = (gs0, gs1, gs2, gs3)
    dsem = (ds0, ds1, ds2, ds3)
    wsem = (ws0, ws1)
    ssem = (ss0, ss1, ss2, ss3)

    @pl.when(wid < NW - 1)
    def _stage_full():
        pltpu.sync_copy(src_hbm.at[pl.ds(wid * C * K, C * K)], src_v)

    @pl.when(wid == NW - 1)
    def _stage_tail():
        ntail = E - (NW - 1) * C * K
        pltpu.sync_copy(src_hbm.at[pl.ds((NW - 1) * C * K, ntail)],
                        src_v.at[pl.ds(0, ntail)])

    class _Dma:
        def __init__(self, src, dst, sem, add=False):
            self.args = (src, dst, sem)
            self.add = add

        def start(self):
            pltpu.async_copy(*self.args, add=self.add)

        def wait(self):
            pltpu.make_async_copy(*self.args).wait()

    def _gather(b, c):
        idx = src_v.at[pl.ds(c * K, K)]
        return _Dma(h_hbm.at[idx], rows[b], gsem[b])

    def _dst_cp(b, c):
        return _Dma(dst_hbm.at[pl.ds((g0 + c) * K, K)], dstb[b], dsem[b])

    def _w_cp(b, c):
        return _Dma(w_hbm.at[pl.ds((g0 + c) * K, K)], wb_[b], wsem[b])

    def _scat(b):
        return _Dma(rows[b], agg_sh.at[dstb[b]], ssem[b], add=True)

    zero = jnp.zeros((16,), jnp.float32)

    def _zrow(r, carry):
        for j in range(D // 16):
            rows0[r, pl.ds(j * 16, 16)] = zero
        return carry

    lax.fori_loop(0, K, _zrow, 0)
    base = tid * RPT
    for k in range(9):
        pltpu.sync_copy(rows0, agg_sh.at[pl.ds(base + k * K, K)])
    pltpu.sync_copy(rows0.at[pl.ds(0, RPT - 9 * K)],
                    agg_sh.at[pl.ds(base + 9 * K, RPT - 9 * K)])
    plsc.subcore_barrier()

    _dst_cp(0, 0).start()
    _dst_cp(1, 1).start()
    _w_cp(0, 0).start()
    _w_cp(1, 1).start()
    _gather(0, 0).start()
    _gather(1, 1).start()

    def _chunk4(c4, carry):
        for sub in range(NB):
            b = sub
            c = c4 * NB + sub
            b2 = (sub + 2) % NB

            @pl.when((c < C - 2) & (g0 + c + 2 < GREAL))
            def _issue(b2=b2, c=c):
                @pl.when(c >= 2)
                def _drain(b2=b2):
                    _scat(b2).wait()
                _dst_cp(b2, c + 2).start()
                _gather(b2, c + 2).start()

            @pl.when(g0 + c < GREAL)
            def _work(b=b, c=c, sub=sub):
                _gather(b, 0).wait()
                _w_cp(b % 2, c).wait()

                def _grp(g, carry2, b=b, sub=sub):
                    wv = wb_[sub % 2][pl.ds(pl.multiple_of(g * 16, 16), 16)]
                    base2 = g * 16
                    for e in range(16):
                        wvb = _lane_bcast(wv, e)
                        r = base2 + e
                        for j in range(D // 16):
                            sl = pl.ds(j * 16, 16)
                            rows[b][r, sl] = rows[b][r, sl] * wvb
                    return carry2

                lax.fori_loop(0, K // 16, _grp, 0)

                @pl.when((c < C - 2) & (g0 + c + 2 < GREAL))
                def _issue_w(c=c, sub=sub):
                    _w_cp(sub % 2, c + 2).start()

                _dst_cp(b, c).wait()
                _scat(b).start()
        return carry

    lax.fori_loop(0, C // NB, _chunk4, 0)
    for b in range(NB):
        _scat(b).wait()
    plsc.subcore_barrier()

    @pl.when(cid == 0)
    def _wb0():
        for k in range(9):
            off = base + k * K
            pltpu.sync_copy(agg_sh.at[pl.ds(off, K)], rows0)
            pltpu.sync_copy(rows0, out0_hbm.at[pl.ds(off, K)])
        tail = RPT - 9 * K
        off = base + 9 * K
        pltpu.sync_copy(agg_sh.at[pl.ds(off, tail)], rows0.at[pl.ds(0, tail)])
        pltpu.sync_copy(rows0.at[pl.ds(0, tail)], out0_hbm.at[pl.ds(off, tail)])

    @pl.when(cid == 1)
    def _wb1():
        for k in range(9):
            off = base + k * K
            pltpu.sync_copy(agg_sh.at[pl.ds(off, K)], rows0)
            pltpu.sync_copy(rows0, out1_hbm.at[pl.ds(off, K)])
        tail = RPT - 9 * K
        off = base + 9 * K
        pltpu.sync_copy(agg_sh.at[pl.ds(off, tail)], rows0.at[pl.ds(0, tail)])
        pltpu.sync_copy(rows0.at[pl.ds(0, tail)], out1_hbm.at[pl.ds(off, tail)])


def kernel(x, edge_index, edge_weight, W1, b1, W2, b2):
    src = edge_index[0].astype(jnp.int32)
    dst = edge_index[1].astype(jnp.int32)
    w = edge_weight.astype(jnp.float32)

    h = _mlp1(x, W1, b1.reshape(1, D))
    p0, p1 = _edge_agg(h, src, dst, w)
    out = _mlp2(x, p0, p1, W2, b2.reshape(1, D))
    return out


# speedup vs baseline: 2.7016x; 1.1386x over previous
import functools

import jax
import jax.numpy as jnp
from jax import lax
from jax.experimental import pallas as pl
from jax.experimental.pallas import tpu as pltpu
from jax.experimental.pallas import tpu_sc as plsc

N = 10000
E = 320000
D = 128

NC = 2
NS = 16
NW = NC * NS
K = 64
C = 160
GREAL = E // K
SROW = 2 * K
NB = 4
NPAD = 10112
RPT = NPAD // NS
BLK = 2000


def _mlp1_body(x_ref, w_ref, b_ref, o_ref):
    acc = jnp.dot(x_ref[...], w_ref[...], preferred_element_type=jnp.float32)
    o_ref[...] = jnp.maximum(acc + b_ref[...], 0.0)


def _mlp1(x, W, b):
    return pl.pallas_call(
        _mlp1_body,
        grid=(N // BLK,),
        in_specs=[
            pl.BlockSpec((BLK, D), lambda i: (i, 0)),
            pl.BlockSpec((D, D), lambda i: (0, 0)),
            pl.BlockSpec((1, D), lambda i: (0, 0)),
        ],
        out_specs=pl.BlockSpec((BLK, D), lambda i: (i, 0)),
        out_shape=jax.ShapeDtypeStruct((N, D), jnp.float32),
    )(x, W, b)


def _mlp2_body(x_ref, p0_ref, p1_ref, w_ref, b_ref, o_ref):
    s = x_ref[...] + p0_ref[...] + p1_ref[...]
    acc = jnp.dot(s, w_ref[...], preferred_element_type=jnp.float32)
    o_ref[...] = jnp.maximum(acc + b_ref[...], 0.0)


def _mlp2(x, p0, p1, W, b):
    return pl.pallas_call(
        _mlp2_body,
        grid=(N // BLK,),
        in_specs=[
            pl.BlockSpec((BLK, D), lambda i: (i, 0)),
            pl.BlockSpec((BLK, D), lambda i: (i, 0)),
            pl.BlockSpec((BLK, D), lambda i: (i, 0)),
            pl.BlockSpec((D, D), lambda i: (0, 0)),
            pl.BlockSpec((1, D), lambda i: (0, 0)),
        ],
        out_specs=pl.BlockSpec((BLK, D), lambda i: (i, 0)),
        out_shape=jax.ShapeDtypeStruct((N, D), jnp.float32),
    )(x, p0, p1, W, b)


def _lane_bcast(vec, lane):
    idx = jnp.full((16, 1), lane, jnp.int32)
    dnums = lax.GatherDimensionNumbers(
        offset_dims=(), collapsed_slice_dims=(0,), start_index_map=(0,))
    return lax.gather(vec, idx, dnums, (1,),
                      mode=lax.GatherScatterMode.PROMISE_IN_BOUNDS)


_MESH = plsc.VectorSubcoreMesh(core_axis_name="c", subcore_axis_name="s")


@functools.partial(
    pl.kernel,
    mesh=_MESH,
    out_type=[jax.ShapeDtypeStruct((NPAD, D), jnp.float32),
              jax.ShapeDtypeStruct((NPAD, D), jnp.float32)],
    scratch_types=[
        pltpu.VMEM_SHARED((NPAD, D), jnp.float32),
        pltpu.VMEM((C * K,), jnp.int32),
        pltpu.VMEM((K,), jnp.int32),
        pltpu.VMEM((K,), jnp.int32),
        pltpu.VMEM((K,), jnp.int32),
        pltpu.VMEM((K,), jnp.int32),
        pltpu.VMEM((K,), jnp.float32),
        pltpu.VMEM((K,), jnp.float32),
        pltpu.VMEM((K, D), jnp.float32),
        pltpu.VMEM((K, D), jnp.float32),
        pltpu.VMEM((K, D), jnp.float32),
        pltpu.VMEM((K, D), jnp.float32),
        pltpu.SemaphoreType.DMA,
        pltpu.SemaphoreType.DMA,
        pltpu.SemaphoreType.DMA,
        pltpu.SemaphoreType.DMA,
        pltpu.SemaphoreType.DMA,
        pltpu.SemaphoreType.DMA,
        pltpu.SemaphoreType.DMA,
        pltpu.SemaphoreType.DMA,
        pltpu.SemaphoreType.DMA,
        pltpu.SemaphoreType.DMA,
        pltpu.SemaphoreType.DMA,
        pltpu.SemaphoreType.DMA,
        pltpu.SemaphoreType.DMA,
        pltpu.SemaphoreType.DMA,
    ],
)
def _edge_agg(h_hbm, edge_hbm, w_hbm, out0_hbm, out1_hbm,
              agg_sh, src_v, dst0, dst1, dst2, dst3, w0, w1,
              rows0, rows1, rows2, rows3,
              gs0, gs1, gs2, gs3, ds0, ds1, ds2, ds3, ws0, ws1,
              ss0, ss1, ss2, ss3):
    cid = lax.axis_index("c")
    tid = lax.axis_index("s")
    wid = cid * NS + tid
    g0 = wid * C

    dstb = (dst0, dst1, dst2, dst3)
    wb_ = (w0, w1)
    rows = (rows0, rows1, rows2, rows3)
    gsem = (gs0, gs1, gs2, gs3)
    dsem = (ds0, ds1, ds2, ds3)
    wsem = (ws0, ws1)
    ssem = (ss0, ss1, ss2, ss3)

    @pl.when(wid < NW - 1)
    def _stage_full():
        pltpu.sync_copy(edge_hbm.at[0, pl.ds(wid * C * K, C * K)], src_v)

    @pl.when(wid == NW - 1)
    def _stage_tail():
        ntail = E - (NW - 1) * C * K
        pltpu.sync_copy(edge_hbm.at[0, pl.ds((NW - 1) * C * K, ntail)],
                        src_v.at[pl.ds(0, ntail)])

    class _Dma:
        def __init__(self, src, dst, sem, add=False):
            self.args = (src, dst, sem)
            self.add = add

        def start(self):
            pltpu.async_copy(*self.args, add=self.add)

        def wait(self):
            pltpu.make_async_copy(*self.args).wait()

    def _gather(b, c):
        idx = src_v.at[pl.ds(c * K, K)]
        return _Dma(h_hbm.at[idx], rows[b], gsem[b])

    def _dst_cp(b, c):
        return _Dma(edge_hbm.at[1, pl.ds((g0 + c) * K, K)], dstb[b], dsem[b])

    def _w_cp(b, c):
        return _Dma(w_hbm.at[pl.ds((g0 + c) * K, K)], wb_[b], wsem[b])

    def _scat(b):
        return _Dma(rows[b], agg_sh.at[dstb[b]], ssem[b], add=True)

    zero = jnp.zeros((16,), jnp.float32)

    def _zrow(r, carry):
        for j in range(D // 16):
            rows0[r, pl.ds(j * 16, 16)] = zero
        return carry

    lax.fori_loop(0, K, _zrow, 0)
    base = tid * RPT
    for k in range(9):
        pltpu.sync_copy(rows0, agg_sh.at[pl.ds(base + k * K, K)])
    pltpu.sync_copy(rows0.at[pl.ds(0, RPT - 9 * K)],
                    agg_sh.at[pl.ds(base + 9 * K, RPT - 9 * K)])
    plsc.subcore_barrier()

    _dst_cp(0, 0).start()
    _dst_cp(1, 1).start()
    _w_cp(0, 0).start()
    _w_cp(1, 1).start()
    _gather(0, 0).start()
    _gather(1, 1).start()

    def _chunk4(c4, carry):
        for sub in range(NB):
            b = sub
            c = c4 * NB + sub
            b2 = (sub + 2) % NB

            @pl.when((c < C - 2) & (g0 + c + 2 < GREAL))
            def _issue(b2=b2, c=c):
                @pl.when(c >= 2)
                def _drain(b2=b2):
                    _scat(b2).wait()
                _dst_cp(b2, c + 2).start()
                _gather(b2, c + 2).start()

            @pl.when(g0 + c < GREAL)
            def _work(b=b, c=c, sub=sub):
                _gather(b, 0).wait()
                _w_cp(b % 2, c).wait()

                def _grp(g, carry2, b=b, sub=sub):
                    wv = wb_[sub % 2][pl.ds(pl.multiple_of(g * 16, 16), 16)]
                    base2 = g * 16
                    for e in range(16):
                        wvb = _lane_bcast(wv, e)
                        r = base2 + e
                        for j in range(D // 16):
                            sl = pl.ds(j * 16, 16)
                            rows[b][r, sl] = rows[b][r, sl] * wvb
                    return carry2

                lax.fori_loop(0, K // 16, _grp, 0)

                @pl.when((c < C - 2) & (g0 + c + 2 < GREAL))
                def _issue_w(c=c, sub=sub):
                    _w_cp(sub % 2, c + 2).start()

                _dst_cp(b, c).wait()
                _scat(b).start()
        return carry

    lax.fori_loop(0, C // NB, _chunk4, 0)
    for b in range(NB):
        _scat(b).wait()
    plsc.subcore_barrier()

    @pl.when(cid == 0)
    def _wb0():
        for k in range(9):
            off = base + k * K
            pltpu.sync_copy(agg_sh.at[pl.ds(off, K)], rows0)
            pltpu.sync_copy(rows0, out0_hbm.at[pl.ds(off, K)])
        tail = RPT - 9 * K
        off = base + 9 * K
        pltpu.sync_copy(agg_sh.at[pl.ds(off, tail)], rows0.at[pl.ds(0, tail)])
        pltpu.sync_copy(rows0.at[pl.ds(0, tail)], out0_hbm.at[pl.ds(off, tail)])

    @pl.when(cid == 1)
    def _wb1():
        for k in range(9):
            off = base + k * K
            pltpu.sync_copy(agg_sh.at[pl.ds(off, K)], rows0)
            pltpu.sync_copy(rows0, out1_hbm.at[pl.ds(off, K)])
        tail = RPT - 9 * K
        off = base + 9 * K
        pltpu.sync_copy(agg_sh.at[pl.ds(off, tail)], rows0.at[pl.ds(0, tail)])
        pltpu.sync_copy(rows0.at[pl.ds(0, tail)], out1_hbm.at[pl.ds(off, tail)])


def kernel(x, edge_index, edge_weight, W1, b1, W2, b2):
    edges = edge_index.astype(jnp.int32)
    w = edge_weight.astype(jnp.float32)

    h = _mlp1(x, W1, b1.reshape(1, D))
    p0, p1 = _edge_agg(h, edges, w)
    out = _mlp2(x, p0, p1, W2, b2.reshape(1, D))
    return out


# speedup vs baseline: 2.7575x; 1.0207x over previous
import functools

import jax
import jax.numpy as jnp
from jax import lax
from jax.experimental import pallas as pl
from jax.experimental.pallas import tpu as pltpu
from jax.experimental.pallas import tpu_sc as plsc

N = 10000
E = 320000
D = 128

NC = 2
NS = 16
NW = NC * NS
K = 64
C = 160
GREAL = E // K
SROW = 2 * K
NB = 4
NPAD = 10112
RPT = NPAD // NS
BLK = 2000


def _mlp1_body(x_ref, w_ref, b_ref, o_ref):
    acc = jnp.dot(x_ref[...], w_ref[...], preferred_element_type=jnp.float32)
    o_ref[...] = jnp.maximum(acc + b_ref[...], 0.0)


def _mlp1(x, W, b):
    return pl.pallas_call(
        _mlp1_body,
        grid=(N // BLK,),
        in_specs=[
            pl.BlockSpec((BLK, D), lambda i: (i, 0)),
            pl.BlockSpec((D, D), lambda i: (0, 0)),
            pl.BlockSpec((1, D), lambda i: (0, 0)),
        ],
        out_specs=pl.BlockSpec((BLK, D), lambda i: (i, 0)),
        out_shape=jax.ShapeDtypeStruct((N, D), jnp.float32),
    )(x, W, b)


def _mlp2_body(x_ref, p0_ref, p1_ref, w_ref, b_ref, o_ref):
    s = x_ref[...] + p0_ref[...] + p1_ref[...]
    acc = jnp.dot(s, w_ref[...], preferred_element_type=jnp.float32)
    o_ref[...] = jnp.maximum(acc + b_ref[...], 0.0)


def _mlp2(x, p0, p1, W, b):
    return pl.pallas_call(
        _mlp2_body,
        grid=(N // BLK,),
        in_specs=[
            pl.BlockSpec((BLK, D), lambda i: (i, 0)),
            pl.BlockSpec((BLK, D), lambda i: (i, 0)),
            pl.BlockSpec((BLK, D), lambda i: (i, 0)),
            pl.BlockSpec((D, D), lambda i: (0, 0)),
            pl.BlockSpec((1, D), lambda i: (0, 0)),
        ],
        out_specs=pl.BlockSpec((BLK, D), lambda i: (i, 0)),
        out_shape=jax.ShapeDtypeStruct((N, D), jnp.float32),
    )(x, p0, p1, W, b)


def _lane_bcast(vec, lane):
    idx = jnp.full((16, 1), lane, jnp.int32)
    dnums = lax.GatherDimensionNumbers(
        offset_dims=(), collapsed_slice_dims=(0,), start_index_map=(0,))
    return lax.gather(vec, idx, dnums, (1,),
                      mode=lax.GatherScatterMode.PROMISE_IN_BOUNDS)


_MESH = plsc.VectorSubcoreMesh(core_axis_name="c", subcore_axis_name="s")


@functools.partial(
    pl.kernel,
    mesh=_MESH,
    out_type=[jax.ShapeDtypeStruct((NPAD, D), jnp.float32),
              jax.ShapeDtypeStruct((NPAD, D), jnp.float32)],
    scratch_types=[
        pltpu.VMEM_SHARED((NPAD, D), jnp.float32),
        pltpu.VMEM((C * K,), jnp.int32),
        pltpu.VMEM((K,), jnp.int32),
        pltpu.VMEM((K,), jnp.int32),
        pltpu.VMEM((K,), jnp.int32),
        pltpu.VMEM((K,), jnp.int32),
        pltpu.VMEM((K,), jnp.float32),
        pltpu.VMEM((K,), jnp.float32),
        pltpu.VMEM((K, D), jnp.float32),
        pltpu.VMEM((K, D), jnp.float32),
        pltpu.VMEM((K, D), jnp.float32),
        pltpu.VMEM((K, D), jnp.float32),
        pltpu.SemaphoreType.DMA,
        pltpu.SemaphoreType.DMA,
        pltpu.SemaphoreType.DMA,
        pltpu.SemaphoreType.DMA,
        pltpu.SemaphoreType.DMA,
        pltpu.SemaphoreType.DMA,
        pltpu.SemaphoreType.DMA,
        pltpu.SemaphoreType.DMA,
        pltpu.SemaphoreType.DMA,
        pltpu.SemaphoreType.DMA,
        pltpu.SemaphoreType.DMA,
        pltpu.SemaphoreType.DMA,
        pltpu.SemaphoreType.DMA,
        pltpu.SemaphoreType.DMA,
    ],
)
def _edge_agg(h_hbm, edge_hbm, w_hbm, out0_hbm, out1_hbm,
              agg_sh, src_v, dst0, dst1, dst2, dst3, w0, w1,
              rows0, rows1, rows2, rows3,
              gs0, gs1, gs2, gs3, ds0, ds1, ds2, ds3, ws0, ws1,
              ss0, ss1, ss2, ss3):
    cid = lax.axis_index("c")
    tid = lax.axis_index("s")
    wid = cid * NS + tid
    g0 = wid * C

    dstb = (dst0, dst1, dst2, dst3)
    wb_ = (w0, w1)
    rows = (rows0, rows1, rows2, rows3)
    gsem = (gs0, gs1, gs2, gs3)
    dsem = (ds0, ds1, ds2, ds3)
    wsem = (ws0, ws1)
    ssem = (ss0, ss1, ss2, ss3)

    @pl.when(wid < NW - 1)
    def _stage_full():
        pltpu.sync_copy(edge_hbm.at[0, pl.ds(wid * C * K, C * K)], src_v)

    @pl.when(wid == NW - 1)
    def _stage_tail():
        ntail = E - (NW - 1) * C * K
        pltpu.sync_copy(edge_hbm.at[0, pl.ds((NW - 1) * C * K, ntail)],
                        src_v.at[pl.ds(0, ntail)])

    class _Dma:
        def __init__(self, src, dst, sem, add=False):
            self.args = (src, dst, sem)
            self.add = add

        def start(self):
            pltpu.async_copy(*self.args, add=self.add)

        def wait(self):
            pltpu.make_async_copy(*self.args).wait()

    def _gather(b, c):
        idx = src_v.at[pl.ds(c * K, K)]
        return _Dma(h_hbm.at[idx], rows[b], gsem[b])

    def _dst_cp(b, c):
        return _Dma(edge_hbm.at[1, pl.ds((g0 + c) * K, K)], dstb[b], dsem[b])

    def _w_cp(b, c):
        return _Dma(w_hbm.at[pl.ds((g0 + c) * K, K)], wb_[b], wsem[b])

    def _scat(b):
        return _Dma(rows[b], agg_sh.at[dstb[b]], ssem[b], add=True)

    zero = jnp.zeros((16,), jnp.float32)

    def _zrow(r, carry):
        for j in range(D // 16):
            rows0[r, pl.ds(j * 16, 16)] = zero
        return carry

    lax.fori_loop(0, K, _zrow, 0)
    base = tid * RPT
    _zsems = (gs0, gs1, gs2, gs3, ds0, ds1, ds2, ds3, ws0, ws1)

    def _zcp(k):
        if k < 9:
            return _Dma(rows0, agg_sh.at[pl.ds(base + k * K, K)], _zsems[k])
        tail = RPT - 9 * K
        return _Dma(rows0.at[pl.ds(0, tail)],
                    agg_sh.at[pl.ds(base + 9 * K, tail)], _zsems[k])

    for k in range(10):
        _zcp(k).start()
    for k in range(10):
        _zcp(k).wait()
    plsc.subcore_barrier()

    _dst_cp(0, 0).start()
    _dst_cp(1, 1).start()
    _w_cp(0, 0).start()
    _w_cp(1, 1).start()
    _gather(0, 0).start()
    _gather(1, 1).start()

    def _chunk4(c4, carry):
        for sub in range(NB):
            b = sub
            c = c4 * NB + sub
            b2 = (sub + 2) % NB

            @pl.when((c < C - 2) & (g0 + c + 2 < GREAL))
            def _issue(b2=b2, c=c):
                @pl.when(c >= 2)
                def _drain(b2=b2):
                    _scat(b2).wait()
                _dst_cp(b2, c + 2).start()
                _gather(b2, c + 2).start()

            @pl.when(g0 + c < GREAL)
            def _work(b=b, c=c, sub=sub):
                _gather(b, 0).wait()
                _w_cp(b % 2, c).wait()

                def _grp(g, carry2, b=b, sub=sub):
                    wv = wb_[sub % 2][pl.ds(pl.multiple_of(g * 16, 16), 16)]
                    base2 = g * 16
                    for e in range(16):
                        wvb = _lane_bcast(wv, e)
                        r = base2 + e
                        for j in range(D // 16):
                            sl = pl.ds(j * 16, 16)
                            rows[b][r, sl] = rows[b][r, sl] * wvb
                    return carry2

                lax.fori_loop(0, K // 16, _grp, 0)

                @pl.when((c < C - 2) & (g0 + c + 2 < GREAL))
                def _issue_w(c=c, sub=sub):
                    _w_cp(sub % 2, c + 2).start()

                _dst_cp(b, c).wait()
                _scat(b).start()
        return carry

    lax.fori_loop(0, C // NB, _chunk4, 0)
    for b in range(NB):
        _scat(b).wait()
    plsc.subcore_barrier()

    _segs = [(base + k * K, K) for k in range(9)] + [
        (base + 9 * K, RPT - 9 * K)]

    def _readback(out_hbm):
        def _rbuf(k):
            n = _segs[k][1]
            r = rows[k % 2]
            return r if n == K else r.at[pl.ds(0, n)]

        def _rin(k):
            off, n = _segs[k]
            return _Dma(agg_sh.at[pl.ds(off, n)], _rbuf(k), gsem[k % 2])

        def _rout(k):
            off, n = _segs[k]
            return _Dma(_rbuf(k), out_hbm.at[pl.ds(off, n)], ssem[k % 2])

        nseg = len(_segs)
        _rin(0).start()
        for k in range(nseg):
            if k + 1 < nseg:
                if k >= 1:
                    _rout(k - 1).wait()
                _rin(k + 1).start()
            _rin(k).wait()
            _rout(k).start()
        _rout(nseg - 2).wait()
        _rout(nseg - 1).wait()

    @pl.when(cid == 0)
    def _wb0():
        _readback(out0_hbm)

    @pl.when(cid == 1)
    def _wb1():
        _readback(out1_hbm)


def kernel(x, edge_index, edge_weight, W1, b1, W2, b2):
    edges = edge_index.astype(jnp.int32)
    w = edge_weight.astype(jnp.float32)

    h = _mlp1(x, W1, b1.reshape(1, D))
    p0, p1 = _edge_agg(h, edges, w)
    out = _mlp2(x, p0, p1, W2, b2.reshape(1, D))
    return out
